# Initial kernel scaffold; baseline (speedup 1.0000x reference)
import jax
import jax.numpy as jnp
from jax.experimental import pallas as pl


def kernel(x1, x2, edges, at_src, at_dst, at_armies, dep_tgt, dep_armies, params):
    raise NotImplementedError("write your pallas kernel here")


# speedup vs baseline: 3.7615x; 3.7615x over previous
import functools

import jax
import jax.numpy as jnp
from jax import lax
from jax.experimental import pallas as pl
from jax.experimental.pallas import tpu as pltpu
from jax.experimental.pallas import tpu_sc as plsc

f32 = jnp.float32
i32 = jnp.int32

N = 20
E = 200
M = 256
NA = 8
ND = 4
NW = 16
L = 16
NORD = float(NA + ND)
EPS = 1e-5

ROW_D = 20
ROW_T = 40
ROW_W1 = 60
ROW_W2 = 61
ROW_WD = 62
ROW_CF = 63
ROW_SV = 64
ROW_C = 65
TBL_ROWS = 72


def _graph_norm_nodes(x, nref):
    w, b, s = nref[0:1, :], nref[1:2, :], nref[2:3, :]
    mean = jnp.mean(x, axis=0, keepdims=True)
    out = x - mean * s
    var = jnp.mean(out * out, axis=0, keepdims=True)
    return w * out / jnp.sqrt(var + EPS) + b


def _tconv(x, W4, b4, Wb, esrc_t, edst_t, valid_row):
    q = jnp.dot(x, W4[0], preferred_element_type=f32) + b4[0:1, :]
    k = jnp.dot(x, W4[1], preferred_element_type=f32) + b4[1:2, :]
    v = jnp.dot(x, W4[2], preferred_element_type=f32) + b4[2:3, :]
    xr = jnp.dot(x, W4[3], preferred_element_type=f32) + b4[3:4, :]
    P = lax.dot_general(q, k, (((1,), (1,)), ((), ())), preferred_element_type=f32)
    s_row = jnp.sum(edst_t * jnp.dot(P, esrc_t, preferred_element_type=f32),
                    axis=0, keepdims=True) * (1.0 / jnp.sqrt(f32(N)))
    edst_b = edst_t > 0.5
    masked = jnp.where(edst_b & valid_row, s_row, f32(-1e30))
    smax = jnp.max(masked, axis=1, keepdims=True)
    smax = jnp.where(smax > f32(-1e29), smax, f32(0.0))
    sm_e = jnp.sum(edst_t * smax, axis=0, keepdims=True)
    ex = jnp.where(valid_row, jnp.exp(s_row - sm_e), f32(0.0))
    den = jnp.sum(edst_t * ex, axis=1, keepdims=True)
    den_e = jnp.sum(edst_t * den, axis=0, keepdims=True)
    alpha = ex / den_e
    vs_t = lax.dot_general(v, esrc_t, (((0,), (0,)), ((), ())),
                           preferred_element_type=f32)
    agg = lax.dot_general(edst_t, alpha * vs_t, (((1,), (1,)), ((), ())),
                          preferred_element_type=f32)
    cat = jnp.concatenate([agg, xr, agg - xr], axis=1)
    beta = jax.nn.sigmoid(jnp.dot(cat, Wb, preferred_element_type=f32))
    return beta * xr + (1.0 - beta) * agg


def _tc1_body(x1_ref, x2_ref, src_row_ref, dst_row_ref, src_col_ref, dst_col_ref,
              W1_ref, b1_ref, Wb1_ref, W2_ref, b2_ref, Wb2_ref,
              n1_ref, n2_ref, n3_ref,
              gateW_ref, gateb_ref, nnW_ref, nnb_ref, lin1W_ref, lin1b_ref,
              Wsrc_ref, Wdst_ref, atkb_ref, Wdep_ref, depb_ref, const5_ref,
              tbl_ref, v_ref):
    x1 = x1_ref[...]
    src_row = src_row_ref[...]
    dst_row = dst_row_ref[...]
    h_col = src_col_ref[...] * N + dst_col_ref[...]
    h_row = src_row * N + dst_row
    eq = h_col == h_row
    ii = lax.broadcasted_iota(i32, (E, E), 0)
    jj = lax.broadcasted_iota(i32, (E, E), 1)
    dup_row = jnp.any(eq & (ii < jj), axis=0, keepdims=True)
    valid_row = ~dup_row
    nodes_col = lax.broadcasted_iota(i32, (N, 1), 0)
    esrc_t = jnp.where(nodes_col == src_row, f32(1.0), f32(0.0))
    edst_t = jnp.where(nodes_col == dst_row, f32(1.0), f32(0.0))

    xa = _graph_norm_nodes(
        jnp.maximum(_tconv(x1, W1_ref[...], b1_ref[...], Wb1_ref[...],
                           esrc_t, edst_t, valid_row), 0.0), n1_ref[...])
    xb = _graph_norm_nodes(
        jnp.maximum(_tconv(jnp.concatenate([x1, xa], axis=1), W2_ref[...],
                           b2_ref[...], Wb2_ref[...], esrc_t, edst_t,
                           valid_row), 0.0), n2_ref[...])
    xc = jnp.concatenate([x1, xa, xb], axis=1)

    gl = jnp.dot(xc, gateW_ref[...], preferred_element_type=f32) + gateb_ref[...]
    gl = gl - jnp.max(gl)
    g = jnp.exp(gl)
    g = g / jnp.sum(g)
    h = jnp.dot(xc, nnW_ref[...], preferred_element_type=f32) + nnb_ref[...]
    xg = jnp.sum(g * h, axis=0, keepdims=True)
    n3 = n3_ref[...]
    mg = jnp.mean(xg, axis=1, keepdims=True)
    outg = xg - mg * n3[2:3, :]
    varg = jnp.mean(outg * outg, axis=1, keepdims=True)
    xgn = n3[0:1, :] * outg / jnp.sqrt(varg + EPS) + n3[1:2, :]
    val = jnp.dot(jnp.concatenate([jnp.maximum(xgn, 0.0), x2_ref[...]], axis=1),
                  lin1W_ref[...], preferred_element_type=f32) + lin1b_ref[...]
    v_ref[...] = jnp.tanh(val)

    mask = x1[:, 5:15] == f32(1.0)
    col_sum = jnp.sum(jnp.where(mask, x1[:, 0:1], 0.0), axis=0, keepdims=True)
    col_cnt = jnp.sum(jnp.where(mask, f32(1.0), f32(0.0)), axis=0, keepdims=True)
    den = jnp.where(col_cnt - 1.0 > 0.0, col_cnt - 1.0, 1.0)
    tmp = jnp.where(mask, (col_sum - x1[:, 0:1]) / den, 0.0)

    X = jnp.concatenate([xa, xb], axis=1)
    F = jnp.concatenate([x1, tmp, X], axis=1)
    Sp = jnp.dot(F, Wsrc_ref[...], preferred_element_type=f32) + atkb_ref[...]
    Dd = jnp.dot(F, Wdst_ref[...], preferred_element_type=f32)
    Tp = jnp.dot(F, Wdep_ref[...], preferred_element_type=f32) + depb_ref[...]
    pick = jnp.where((lax.broadcasted_iota(i32, (1, 15), 1) == 3)
                     | (lax.broadcasted_iota(i32, (1, 15), 1) == 4),
                     f32(1.0), f32(0.0))
    c_row = lax.dot_general(pick, x1, (((1,), (1,)), ((), ())),
                            preferred_element_type=f32)
    tbl_ref[...] = jnp.concatenate(
        [Sp, Dd, Tp, const5_ref[...], c_row,
         jnp.zeros((TBL_ROWS - ROW_C - 1, N), f32)], axis=0)


def _tc2_body(p_ref, o_ref):
    x = p_ref[...]
    m = jnp.max(x)
    ex = jnp.exp(x - m)
    o_ref[...] = x - m - jnp.log(jnp.sum(ex))


def _sc_rsqrt(x):
    i = plsc.bitcast(x, i32)
    i = jnp.int32(0x5F3759DF) - lax.shift_right_logical(i, 1)
    y = plsc.bitcast(i, f32)
    for _ in range(3):
        y = y * (1.5 - 0.5 * x * y * y)
    return y


@functools.cache
def _sc_moves_kernel():
    mesh = plsc.VectorSubcoreMesh(core_axis_name="c", subcore_axis_name="s",
                                  num_cores=2, num_subcores=16)
    return pl.kernel(
        _sc_moves_body,
        out_type=jax.ShapeDtypeStruct((NW, L), f32),
        mesh=mesh,
        compiler_params=pltpu.CompilerParams(needs_layout_passes=False),
        scratch_types=[
            pltpu.VMEM((TBL_ROWS, N), f32),
            pltpu.VMEM((NA, L), i32),
            pltpu.VMEM((NA, L), i32),
            pltpu.VMEM((NA, L), f32),
            pltpu.VMEM((ND, L), i32),
            pltpu.VMEM((ND, L), f32),
            pltpu.VMEM((L,), f32),
            pltpu.SemaphoreType.DMA,
        ],
    )


def _sc_moves_body(tbl_hbm, asrc_hbm, adst_hbm, aarm_hbm, dtgt_hbm, darm_hbm,
                   p_hbm, tbl_v, asrc_v, adst_v, aarm_v, dtgt_v, darm_v, out_v, sem):
    wid = lax.axis_index("s") * 2 + lax.axis_index("c")

    @pl.when(wid < NW)
    def _():
        copies = [
            pltpu.async_copy(tbl_hbm, tbl_v, sem),
            pltpu.async_copy(asrc_hbm.at[wid], asrc_v, sem),
            pltpu.async_copy(adst_hbm.at[wid], adst_v, sem),
            pltpu.async_copy(aarm_hbm.at[wid], aarm_v, sem),
            pltpu.async_copy(dtgt_hbm.at[wid], dtgt_v, sem),
            pltpu.async_copy(darm_hbm.at[wid], darm_v, sem),
        ]
        for c in copies:
            c.wait()

        def splat(v):
            return jnp.full((L,), v, i32)

        gath = functools.partial(plsc.load_gather, tbl_v)
        asrc = [asrc_v[j] for j in range(NA)]
        adst = [adst_v[j] for j in range(NA)]
        arms = [aarm_v[j] for j in range(NA)]
        dtgt = [dtgt_v[j] for j in range(ND)]
        darm = [darm_v[j] for j in range(ND)]
        row_c = splat(ROW_C)
        es = [0.6 * arms[j] - 0.7 * gath([row_c, adst[j]]) for j in range(NA)]

        p = jnp.zeros((L,), f32)
        for f in range(N):
            fcol = splat(f)
            w1f = gath([splat(ROW_W1), fcol])
            w2f = gath([splat(ROW_W2), fcol])
            wdf = gath([splat(ROW_WD), fcol])
            cff = gath([splat(ROW_CF), fcol])
            svf = gath([splat(ROW_SV), fcol])
            t1 = jnp.zeros((L,), f32)
            t2 = jnp.zeros((L,), f32)
            for j in range(NA):
                o = (gath([asrc[j], fcol]) + gath([adst[j] + ROW_D, fcol])
                     + arms[j] * w1f + es[j] * w2f)
                t1 += o
                t2 += o * o
            for j in range(ND):
                o = gath([dtgt[j] + ROW_T, fcol]) + darm[j] * wdf
                t1 += o
                t2 += o * o
            mu = t1 * (1.0 / NORD)
            var = t2 * (1.0 / NORD) - mu * mu * svf + EPS
            p = p + cff * t1 * _sc_rsqrt(var)
        out_v[...] = p
        pltpu.sync_copy(out_v, p_hbm.at[wid])


def _move_blocks(a):
    k = a.shape[1]
    return a.T.reshape(k, NW, L).transpose(1, 0, 2)


def kernel(x1, x2, edges, at_src, at_dst, at_armies, dep_tgt, dep_armies, params):
    x1 = x1.astype(f32)
    edges = edges.astype(i32)
    src_row = jnp.concatenate([edges[0], edges[1]]).reshape(1, E)
    dst_row = jnp.concatenate([edges[1], edges[0]]).reshape(1, E)

    p = params
    g1, g2 = p["g1"], p["g2"]
    W1 = jnp.stack([g1["Wq"], g1["Wk"], g1["Wv"], g1["Ws"]])
    b1 = jnp.stack([g1["bq"], g1["bk"], g1["bv"], g1["bs"]])
    W2 = jnp.stack([g2["Wq"], g2["Wk"], g2["Wv"], g2["Ws"]])
    b2 = jnp.stack([g2["bq"], g2["bk"], g2["bv"], g2["bs"]])
    n1 = jnp.stack([p["n1"]["w"], p["n1"]["b"], p["n1"]["s"]])
    n2 = jnp.stack([p["n2"]["w"], p["n2"]["b"], p["n2"]["s"]])
    n3 = jnp.stack([p["n3"]["w"], p["n3"]["b"], p["n3"]["s"]])
    atk_W = p["atk_W"]
    Wsrc = jnp.concatenate([atk_W[0:15], atk_W[30:40], atk_W[50:90]], axis=0)
    Wdst = jnp.concatenate([atk_W[15:30], atk_W[40:50], atk_W[90:130]], axis=0)
    dep_W = p["dep_W"]
    acc_w = p["acc_W"][:, 0]
    n4w, n4b, n4s = p["n4"]["w"], p["n4"]["b"], p["n4"]["s"]
    cf = n4w * acc_w * (1.0 - n4s)
    sv = n4s * (2.0 - n4s)
    const5 = jnp.stack([atk_W[130], atk_W[131], dep_W[65], cf, sv])

    tbl, v = pl.pallas_call(
        _tc1_body,
        out_shape=[
            jax.ShapeDtypeStruct((TBL_ROWS, N), f32),
            jax.ShapeDtypeStruct((1, 1), f32),
        ],
    )(x1, x2.astype(f32).reshape(1, 4), src_row, dst_row,
      src_row.reshape(E, 1), dst_row.reshape(E, 1),
      W1, b1, g1["Wb"], W2, b2, g2["Wb"], n1, n2, n3,
      p["gate_W"], p["gate_b"].reshape(1, 1), p["nn_W"],
      p["nn_b"].reshape(1, 20), p["lin1_W"], p["lin1_b"].reshape(1, 1),
      Wsrc, Wdst, p["atk_b"].reshape(1, 20), dep_W[0:65],
      p["dep_b"].reshape(1, 20), const5)

    pv = _sc_moves_kernel()(tbl,
                   _move_blocks(at_src.astype(i32)),
                   _move_blocks(at_dst.astype(i32)),
                   _move_blocks(at_armies.astype(f32)),
                   _move_blocks(dep_tgt.astype(i32)),
                   _move_blocks(dep_armies.astype(f32)))

    logp = pl.pallas_call(
        _tc2_body,
        out_shape=jax.ShapeDtypeStruct((2, 128), f32),
    )(pv.reshape(2, 128))
    return v.reshape(()), logp.reshape(M)


# speedup vs baseline: 3.7670x; 1.0015x over previous
import functools

import jax
import jax.numpy as jnp
from jax import lax
from jax.experimental import pallas as pl
from jax.experimental.pallas import tpu as pltpu
from jax.experimental.pallas import tpu_sc as plsc

f32 = jnp.float32
i32 = jnp.int32

N = 20
E = 200
M = 256
NA = 8
ND = 4
NW = 16
L = 16
NORD = float(NA + ND)
EPS = 1e-5

ROW_D = 20
ROW_T = 40
ROW_W1 = 60
ROW_W2 = 61
ROW_WD = 62
ROW_CF = 63
ROW_SV = 64
ROW_C = 65
TBL_ROWS = 72


def _graph_norm_nodes(x, nref):
    w, b, s = nref[0:1, :], nref[1:2, :], nref[2:3, :]
    mean = jnp.mean(x, axis=0, keepdims=True)
    out = x - mean * s
    var = jnp.mean(out * out, axis=0, keepdims=True)
    return w * out / jnp.sqrt(var + EPS) + b


def _tconv(x, W4, b4, Wb, esrc_t, edst_t, valid_row):
    q = jnp.dot(x, W4[0], preferred_element_type=f32) + b4[0:1, :]
    k = jnp.dot(x, W4[1], preferred_element_type=f32) + b4[1:2, :]
    v = jnp.dot(x, W4[2], preferred_element_type=f32) + b4[2:3, :]
    xr = jnp.dot(x, W4[3], preferred_element_type=f32) + b4[3:4, :]
    P = lax.dot_general(q, k, (((1,), (1,)), ((), ())), preferred_element_type=f32)
    s_row = jnp.sum(edst_t * jnp.dot(P, esrc_t, preferred_element_type=f32),
                    axis=0, keepdims=True) * (1.0 / jnp.sqrt(f32(N)))
    edst_b = edst_t > 0.5
    masked = jnp.where(edst_b & valid_row, s_row, f32(-1e30))
    smax = jnp.max(masked, axis=1, keepdims=True)
    smax = jnp.where(smax > f32(-1e29), smax, f32(0.0))
    sm_e = jnp.sum(edst_t * smax, axis=0, keepdims=True)
    ex = jnp.where(valid_row, jnp.exp(s_row - sm_e), f32(0.0))
    den = jnp.sum(edst_t * ex, axis=1, keepdims=True)
    den_e = jnp.sum(edst_t * den, axis=0, keepdims=True)
    alpha = ex / den_e
    vs_t = lax.dot_general(v, esrc_t, (((0,), (0,)), ((), ())),
                           preferred_element_type=f32)
    agg = lax.dot_general(edst_t, alpha * vs_t, (((1,), (1,)), ((), ())),
                          preferred_element_type=f32)
    cat = jnp.concatenate([agg, xr, agg - xr], axis=1)
    beta = jax.nn.sigmoid(jnp.dot(cat, Wb, preferred_element_type=f32))
    return beta * xr + (1.0 - beta) * agg


def _tc1_body(x1_ref, x2_ref, src_row_ref, dst_row_ref, src_col_ref, dst_col_ref,
              W1_ref, b1_ref, Wb1_ref, W2_ref, b2_ref, Wb2_ref,
              n1_ref, n2_ref, n3_ref,
              gateW_ref, gateb_ref, nnW_ref, nnb_ref, lin1W_ref, lin1b_ref,
              Wsrc_ref, Wdst_ref, atkb_ref, Wdep_ref, depb_ref, const5_ref,
              tbl_ref, v_ref):
    x1 = x1_ref[...]
    src_row = src_row_ref[...]
    dst_row = dst_row_ref[...]
    h_col = src_col_ref[...] * N + dst_col_ref[...]
    h_row = src_row * N + dst_row
    eq = h_col == h_row
    ii = lax.broadcasted_iota(i32, (E, E), 0)
    jj = lax.broadcasted_iota(i32, (E, E), 1)
    dup_row = jnp.any(eq & (ii < jj), axis=0, keepdims=True)
    valid_row = ~dup_row
    nodes_col = lax.broadcasted_iota(i32, (N, 1), 0)
    esrc_t = jnp.where(nodes_col == src_row, f32(1.0), f32(0.0))
    edst_t = jnp.where(nodes_col == dst_row, f32(1.0), f32(0.0))

    xa = _graph_norm_nodes(
        jnp.maximum(_tconv(x1, W1_ref[...], b1_ref[...], Wb1_ref[...],
                           esrc_t, edst_t, valid_row), 0.0), n1_ref[...])
    xb = _graph_norm_nodes(
        jnp.maximum(_tconv(jnp.concatenate([x1, xa], axis=1), W2_ref[...],
                           b2_ref[...], Wb2_ref[...], esrc_t, edst_t,
                           valid_row), 0.0), n2_ref[...])
    xc = jnp.concatenate([x1, xa, xb], axis=1)

    gl = jnp.dot(xc, gateW_ref[...], preferred_element_type=f32) + gateb_ref[...]
    gl = gl - jnp.max(gl)
    g = jnp.exp(gl)
    g = g / jnp.sum(g)
    h = jnp.dot(xc, nnW_ref[...], preferred_element_type=f32) + nnb_ref[...]
    xg = jnp.sum(g * h, axis=0, keepdims=True)
    n3 = n3_ref[...]
    mg = jnp.mean(xg, axis=1, keepdims=True)
    outg = xg - mg * n3[2:3, :]
    varg = jnp.mean(outg * outg, axis=1, keepdims=True)
    xgn = n3[0:1, :] * outg / jnp.sqrt(varg + EPS) + n3[1:2, :]
    val = jnp.dot(jnp.concatenate([jnp.maximum(xgn, 0.0), x2_ref[...]], axis=1),
                  lin1W_ref[...], preferred_element_type=f32) + lin1b_ref[...]
    v_ref[...] = jnp.tanh(val)

    mask = x1[:, 5:15] == f32(1.0)
    col_sum = jnp.sum(jnp.where(mask, x1[:, 0:1], 0.0), axis=0, keepdims=True)
    col_cnt = jnp.sum(jnp.where(mask, f32(1.0), f32(0.0)), axis=0, keepdims=True)
    den = jnp.where(col_cnt - 1.0 > 0.0, col_cnt - 1.0, 1.0)
    tmp = jnp.where(mask, (col_sum - x1[:, 0:1]) / den, 0.0)

    X = jnp.concatenate([xa, xb], axis=1)
    F = jnp.concatenate([x1, tmp, X], axis=1)
    Sp = jnp.dot(F, Wsrc_ref[...], preferred_element_type=f32) + atkb_ref[...]
    Dd = jnp.dot(F, Wdst_ref[...], preferred_element_type=f32)
    Tp = jnp.dot(F, Wdep_ref[...], preferred_element_type=f32) + depb_ref[...]
    pick = jnp.where((lax.broadcasted_iota(i32, (1, 15), 1) == 3)
                     | (lax.broadcasted_iota(i32, (1, 15), 1) == 4),
                     f32(1.0), f32(0.0))
    c_row = lax.dot_general(pick, x1, (((1,), (1,)), ((), ())),
                            preferred_element_type=f32)
    tbl_ref[...] = jnp.concatenate(
        [Sp, Dd, Tp, const5_ref[...], c_row,
         jnp.zeros((TBL_ROWS - ROW_C - 1, N), f32)], axis=0)


def _sc_rsqrt(x):
    i = plsc.bitcast(x, i32)
    i = jnp.int32(0x5F3759DF) - lax.shift_right_logical(i, 1)
    y = plsc.bitcast(i, f32)
    for _ in range(3):
        y = y * (1.5 - 0.5 * x * y * y)
    return y


@functools.cache
def _sc_moves_kernel():
    mesh = plsc.VectorSubcoreMesh(core_axis_name="c", subcore_axis_name="s",
                                  num_cores=2, num_subcores=16)
    return pl.kernel(
        _sc_moves_body,
        out_type=jax.ShapeDtypeStruct((NW, L), f32),
        mesh=mesh,
        compiler_params=pltpu.CompilerParams(needs_layout_passes=False),
        scratch_types=[
            pltpu.VMEM((TBL_ROWS, N), f32),
            pltpu.VMEM((NA, L), i32),
            pltpu.VMEM((NA, L), i32),
            pltpu.VMEM((NA, L), f32),
            pltpu.VMEM((ND, L), i32),
            pltpu.VMEM((ND, L), f32),
            pltpu.VMEM((L,), f32),
            pltpu.VMEM((NW, L), f32),
            pltpu.HBM((NW, L), f32),
            pltpu.SemaphoreType.DMA,
        ],
    )


def _sc_moves_body(tbl_hbm, asrc_hbm, adst_hbm, aarm_hbm, dtgt_hbm, darm_hbm,
                   p_hbm, tbl_v, asrc_v, adst_v, aarm_v, dtgt_v, darm_v, out_v,
                   all_v, stage_hbm, sem):
    wid = lax.axis_index("s")

    @pl.when(lax.axis_index("c") == 0)
    def _():
        copies = [
            pltpu.async_copy(tbl_hbm, tbl_v, sem),
            pltpu.async_copy(asrc_hbm.at[wid], asrc_v, sem),
            pltpu.async_copy(adst_hbm.at[wid], adst_v, sem),
            pltpu.async_copy(aarm_hbm.at[wid], aarm_v, sem),
            pltpu.async_copy(dtgt_hbm.at[wid], dtgt_v, sem),
            pltpu.async_copy(darm_hbm.at[wid], darm_v, sem),
        ]
        for c in copies:
            c.wait()

        def splat(v):
            return jnp.full((L,), v, i32)

        gath = functools.partial(plsc.load_gather, tbl_v)
        asrc = [asrc_v[j] for j in range(NA)]
        adst = [adst_v[j] for j in range(NA)]
        arms = [aarm_v[j] for j in range(NA)]
        dtgt = [dtgt_v[j] for j in range(ND)]
        darm = [darm_v[j] for j in range(ND)]
        row_c = splat(ROW_C)
        es = [0.6 * arms[j] - 0.7 * gath([row_c, adst[j]]) for j in range(NA)]

        p = jnp.zeros((L,), f32)
        for f in range(N):
            fcol = splat(f)
            w1f = gath([splat(ROW_W1), fcol])
            w2f = gath([splat(ROW_W2), fcol])
            wdf = gath([splat(ROW_WD), fcol])
            cff = gath([splat(ROW_CF), fcol])
            svf = gath([splat(ROW_SV), fcol])
            t1 = jnp.zeros((L,), f32)
            t2 = jnp.zeros((L,), f32)
            for j in range(NA):
                o = (gath([asrc[j], fcol]) + gath([adst[j] + ROW_D, fcol])
                     + arms[j] * w1f + es[j] * w2f)
                t1 += o
                t2 += o * o
            for j in range(ND):
                o = gath([dtgt[j] + ROW_T, fcol]) + darm[j] * wdf
                t1 += o
                t2 += o * o
            mu = t1 * (1.0 / NORD)
            var = t2 * (1.0 / NORD) - mu * mu * svf + EPS
            p = p + cff * t1 * _sc_rsqrt(var)

        out_v[...] = p
        pltpu.sync_copy(out_v, stage_hbm.at[wid])
        plsc.subcore_barrier()
        pltpu.sync_copy(stage_hbm, all_v)
        rows = [all_v[k] for k in range(NW)]
        mx = rows[0]
        for k in range(1, NW):
            mx = jnp.maximum(mx, rows[k])
        m = jnp.max(mx)
        se = jnp.zeros((L,), f32)
        for k in range(NW):
            se += jnp.exp(rows[k] - m)
        s = jnp.sum(se)
        sv = jnp.full((L,), 1.0, f32) * s
        e = lax.shift_right_logical(plsc.bitcast(sv, i32), 23) - 127
        y = (e.astype(f32) + 0.5) * jnp.float32(0.6931472)
        for _ in range(4):
            y = y + sv * jnp.exp(-y) - 1.0
        out_v[...] = p - m - y
        pltpu.sync_copy(out_v, p_hbm.at[wid])


def _move_blocks(a):
    k = a.shape[1]
    return a.T.reshape(k, NW, L).transpose(1, 0, 2)


def kernel(x1, x2, edges, at_src, at_dst, at_armies, dep_tgt, dep_armies, params):
    x1 = x1.astype(f32)
    edges = edges.astype(i32)
    src_row = jnp.concatenate([edges[0], edges[1]]).reshape(1, E)
    dst_row = jnp.concatenate([edges[1], edges[0]]).reshape(1, E)

    p = params
    g1, g2 = p["g1"], p["g2"]
    W1 = jnp.stack([g1["Wq"], g1["Wk"], g1["Wv"], g1["Ws"]])
    b1 = jnp.stack([g1["bq"], g1["bk"], g1["bv"], g1["bs"]])
    W2 = jnp.stack([g2["Wq"], g2["Wk"], g2["Wv"], g2["Ws"]])
    b2 = jnp.stack([g2["bq"], g2["bk"], g2["bv"], g2["bs"]])
    n1 = jnp.stack([p["n1"]["w"], p["n1"]["b"], p["n1"]["s"]])
    n2 = jnp.stack([p["n2"]["w"], p["n2"]["b"], p["n2"]["s"]])
    n3 = jnp.stack([p["n3"]["w"], p["n3"]["b"], p["n3"]["s"]])
    atk_W = p["atk_W"]
    Wsrc = jnp.concatenate([atk_W[0:15], atk_W[30:40], atk_W[50:90]], axis=0)
    Wdst = jnp.concatenate([atk_W[15:30], atk_W[40:50], atk_W[90:130]], axis=0)
    dep_W = p["dep_W"]
    acc_w = p["acc_W"][:, 0]
    n4w, n4b, n4s = p["n4"]["w"], p["n4"]["b"], p["n4"]["s"]
    cf = n4w * acc_w * (1.0 - n4s)
    sv = n4s * (2.0 - n4s)
    const5 = jnp.stack([atk_W[130], atk_W[131], dep_W[65], cf, sv])

    tbl, v = pl.pallas_call(
        _tc1_body,
        out_shape=[
            jax.ShapeDtypeStruct((TBL_ROWS, N), f32),
            jax.ShapeDtypeStruct((1, 1), f32),
        ],
    )(x1, x2.astype(f32).reshape(1, 4), src_row, dst_row,
      src_row.reshape(E, 1), dst_row.reshape(E, 1),
      W1, b1, g1["Wb"], W2, b2, g2["Wb"], n1, n2, n3,
      p["gate_W"], p["gate_b"].reshape(1, 1), p["nn_W"],
      p["nn_b"].reshape(1, 20), p["lin1_W"], p["lin1_b"].reshape(1, 1),
      Wsrc, Wdst, p["atk_b"].reshape(1, 20), dep_W[0:65],
      p["dep_b"].reshape(1, 20), const5)

    logp = _sc_moves_kernel()(tbl,
                              _move_blocks(at_src.astype(i32)),
                              _move_blocks(at_dst.astype(i32)),
                              _move_blocks(at_armies.astype(f32)),
                              _move_blocks(dep_tgt.astype(i32)),
                              _move_blocks(dep_armies.astype(f32)))
    return v.reshape(()), logp.reshape(M)


# speedup vs baseline: 4.3822x; 1.1633x over previous
import functools

import jax
import jax.numpy as jnp
from jax import lax
from jax.experimental import pallas as pl
from jax.experimental.pallas import tpu as pltpu
from jax.experimental.pallas import tpu_sc as plsc

f32 = jnp.float32
i32 = jnp.int32

N = 20
E = 200
M = 256
NA = 8
ND = 4
NW = 16
L = 16
NORD = float(NA + ND)
EPS = 1e-5

ROW_D = 20
ROW_T = 40
ROW_W1 = 60
ROW_W2 = 61
ROW_WD = 62
ROW_CF = 63
ROW_SV = 64
ROW_C = 65
TBL_ROWS = 72


def _graph_norm_nodes(x, w, b, s):
    mean = jnp.mean(x, axis=0, keepdims=True)
    out = x - mean * s
    var = jnp.mean(out * out, axis=0, keepdims=True)
    return w * out / jnp.sqrt(var + EPS) + b


def _tconv(x, Wq, bq, Wk, bk, Wv, bv, Ws, bs, Wb, esrc_t, edst_t, valid_row):
    q = jnp.dot(x, Wq, preferred_element_type=f32) + bq
    k = jnp.dot(x, Wk, preferred_element_type=f32) + bk
    v = jnp.dot(x, Wv, preferred_element_type=f32) + bv
    xr = jnp.dot(x, Ws, preferred_element_type=f32) + bs
    P = lax.dot_general(q, k, (((1,), (1,)), ((), ())), preferred_element_type=f32)
    s_row = jnp.sum(edst_t * jnp.dot(P, esrc_t, preferred_element_type=f32),
                    axis=0, keepdims=True) * (1.0 / jnp.sqrt(f32(N)))
    edst_b = edst_t > 0.5
    masked = jnp.where(edst_b & valid_row, s_row, f32(-1e30))
    smax = jnp.max(masked, axis=1, keepdims=True)
    smax = jnp.where(smax > f32(-1e29), smax, f32(0.0))
    sm_e = jnp.sum(edst_t * smax, axis=0, keepdims=True)
    ex = jnp.where(valid_row, jnp.exp(s_row - sm_e), f32(0.0))
    den = jnp.sum(edst_t * ex, axis=1, keepdims=True)
    den_e = jnp.sum(edst_t * den, axis=0, keepdims=True)
    alpha = ex / den_e
    vs_t = lax.dot_general(v, esrc_t, (((0,), (0,)), ((), ())),
                           preferred_element_type=f32)
    agg = lax.dot_general(edst_t, alpha * vs_t, (((1,), (1,)), ((), ())),
                          preferred_element_type=f32)
    cat = jnp.concatenate([agg, xr, agg - xr], axis=1)
    beta = jax.nn.sigmoid(jnp.dot(cat, Wb, preferred_element_type=f32))
    return beta * xr + (1.0 - beta) * agg


def _tc1_body(x1_ref, x2_ref, src_row_ref, dst_row_ref, src_col_ref, dst_col_ref,
              q1_ref, bq1_ref, k1_ref, bk1_ref, v1_ref, bv1_ref, s1_ref, bs1_ref, Wb1_ref,
              q2_ref, bq2_ref, k2_ref, bk2_ref, v2_ref, bv2_ref, s2_ref, bs2_ref, Wb2_ref,
              n1w_ref, n1b_ref, n1s_ref, n2w_ref, n2b_ref, n2s_ref,
              n3w_ref, n3b_ref, n3s_ref,
              gateW_ref, gateb_ref, nnW_ref, nnb_ref, lin1W_ref, lin1b_ref,
              atkW_ref, atkb_ref, depW_ref, depb_ref,
              accW_ref, n4w_ref, n4s_ref,
              tbl_ref, v_ref):
    x1 = x1_ref[...]
    src_row = src_row_ref[...]
    dst_row = dst_row_ref[...]
    h_col = src_col_ref[...] * N + dst_col_ref[...]
    h_row = src_row * N + dst_row
    eq = h_col == h_row
    ii = lax.broadcasted_iota(i32, (E, E), 0)
    jj = lax.broadcasted_iota(i32, (E, E), 1)
    dup_row = jnp.any(eq & (ii < jj), axis=0, keepdims=True)
    valid_row = ~dup_row
    nodes_col = lax.broadcasted_iota(i32, (N, 1), 0)
    esrc_t = jnp.where(nodes_col == src_row, f32(1.0), f32(0.0))
    edst_t = jnp.where(nodes_col == dst_row, f32(1.0), f32(0.0))

    xa = _graph_norm_nodes(
        jnp.maximum(_tconv(x1, q1_ref[...], bq1_ref[...], k1_ref[...],
                           bk1_ref[...], v1_ref[...], bv1_ref[...], s1_ref[...],
                           bs1_ref[...], Wb1_ref[...], esrc_t, edst_t,
                           valid_row), 0.0),
        n1w_ref[...], n1b_ref[...], n1s_ref[...])
    xb = _graph_norm_nodes(
        jnp.maximum(_tconv(jnp.concatenate([x1, xa], axis=1), q2_ref[...],
                           bq2_ref[...], k2_ref[...], bk2_ref[...], v2_ref[...],
                           bv2_ref[...], s2_ref[...], bs2_ref[...], Wb2_ref[...],
                           esrc_t, edst_t, valid_row), 0.0),
        n2w_ref[...], n2b_ref[...], n2s_ref[...])
    xc = jnp.concatenate([x1, xa, xb], axis=1)

    gl = jnp.dot(xc, gateW_ref[...], preferred_element_type=f32) + gateb_ref[...]
    gl = gl - jnp.max(gl)
    g = jnp.exp(gl)
    g = g / jnp.sum(g)
    h = jnp.dot(xc, nnW_ref[...], preferred_element_type=f32) + nnb_ref[...]
    xg = jnp.sum(g * h, axis=0, keepdims=True)
    mg = jnp.mean(xg, axis=1, keepdims=True)
    outg = xg - mg * n3s_ref[...]
    varg = jnp.mean(outg * outg, axis=1, keepdims=True)
    xgn = n3w_ref[...] * outg / jnp.sqrt(varg + EPS) + n3b_ref[...]
    val = jnp.dot(jnp.concatenate([jnp.maximum(xgn, 0.0), x2_ref[...]], axis=1),
                  lin1W_ref[...], preferred_element_type=f32) + lin1b_ref[...]
    v_ref[...] = jnp.tanh(val)

    mask = x1[:, 5:15] == f32(1.0)
    col_sum = jnp.sum(jnp.where(mask, x1[:, 0:1], 0.0), axis=0, keepdims=True)
    col_cnt = jnp.sum(jnp.where(mask, f32(1.0), f32(0.0)), axis=0, keepdims=True)
    den = jnp.where(col_cnt - 1.0 > 0.0, col_cnt - 1.0, 1.0)
    tmp = jnp.where(mask, (col_sum - x1[:, 0:1]) / den, 0.0)

    X = jnp.concatenate([xa, xb], axis=1)
    F = jnp.concatenate([x1, tmp, X], axis=1)
    atkW = atkW_ref[...]
    depW = depW_ref[...]
    Wsrc = jnp.concatenate([atkW[0:15], atkW[30:40], atkW[50:90]], axis=0)
    Wdst = jnp.concatenate([atkW[15:30], atkW[40:50], atkW[90:130]], axis=0)
    Sp = jnp.dot(F, Wsrc, preferred_element_type=f32) + atkb_ref[...]
    Dd = jnp.dot(F, Wdst, preferred_element_type=f32)
    Tp = jnp.dot(F, depW[0:65], preferred_element_type=f32) + depb_ref[...]
    pick = jnp.where((lax.broadcasted_iota(i32, (1, 15), 1) == 3)
                     | (lax.broadcasted_iota(i32, (1, 15), 1) == 4),
                     f32(1.0), f32(0.0))
    c_row = lax.dot_general(pick, x1, (((1,), (1,)), ((), ())),
                            preferred_element_type=f32)
    acc = accW_ref[...]
    n4w = n4w_ref[...]
    n4s = n4s_ref[...]
    cf_row = n4w * acc * (1.0 - n4s)
    sv_row = n4s * (2.0 - n4s)
    tbl_ref[...] = jnp.concatenate(
        [Sp, Dd, Tp, atkW[130:131], atkW[131:132], depW[65:66], cf_row, sv_row,
         c_row, jnp.zeros((TBL_ROWS - ROW_C - 1, N), f32)], axis=0)


def _sc_rsqrt(x):
    i = plsc.bitcast(x, i32)
    i = jnp.int32(0x5F3759DF) - lax.shift_right_logical(i, 1)
    y = plsc.bitcast(i, f32)
    for _ in range(3):
        y = y * (1.5 - 0.5 * x * y * y)
    return y


@functools.cache
def _sc_moves_kernel():
    mesh = plsc.VectorSubcoreMesh(core_axis_name="c", subcore_axis_name="s",
                                  num_cores=2, num_subcores=16)
    return pl.kernel(
        _sc_moves_body,
        out_type=jax.ShapeDtypeStruct((NW, L), f32),
        mesh=mesh,
        compiler_params=pltpu.CompilerParams(needs_layout_passes=False),
        scratch_types=[
            pltpu.VMEM((TBL_ROWS, N), f32),
            pltpu.VMEM((L, NA), i32),
            pltpu.VMEM((L, NA), i32),
            pltpu.VMEM((L, NA), f32),
            pltpu.VMEM((L, ND), i32),
            pltpu.VMEM((L, ND), f32),
            pltpu.VMEM((L,), f32),
            pltpu.VMEM((NW, L), f32),
            pltpu.HBM((NW, L), f32),
            pltpu.SemaphoreType.DMA,
        ],
    )


def _sc_moves_body(tbl_hbm, asrc_hbm, adst_hbm, aarm_hbm, dtgt_hbm, darm_hbm,
                   p_hbm, tbl_v, asrc_v, adst_v, aarm_v, dtgt_v, darm_v, out_v,
                   all_v, stage_hbm, sem):
    wid = lax.axis_index("s")

    @pl.when(lax.axis_index("c") == 0)
    def _():
        rows = pl.ds(wid * L, L)
        copies = [
            pltpu.async_copy(tbl_hbm, tbl_v, sem),
            pltpu.async_copy(asrc_hbm.at[rows], asrc_v, sem),
            pltpu.async_copy(adst_hbm.at[rows], adst_v, sem),
            pltpu.async_copy(aarm_hbm.at[rows], aarm_v, sem),
            pltpu.async_copy(dtgt_hbm.at[rows], dtgt_v, sem),
            pltpu.async_copy(darm_hbm.at[rows], darm_v, sem),
        ]
        for c in copies:
            c.wait()

        def splat(v):
            return jnp.full((L,), v, i32)

        lane = jnp.arange(L, dtype=i32)
        gtab = functools.partial(plsc.load_gather, tbl_v)
        asrc = [plsc.load_gather(asrc_v, [lane, splat(j)]) for j in range(NA)]
        adst = [plsc.load_gather(adst_v, [lane, splat(j)]) for j in range(NA)]
        arms = [plsc.load_gather(aarm_v, [lane, splat(j)]) for j in range(NA)]
        dtgt = [plsc.load_gather(dtgt_v, [lane, splat(j)]) for j in range(ND)]
        darm = [plsc.load_gather(darm_v, [lane, splat(j)]) for j in range(ND)]
        row_c = splat(ROW_C)
        es = [0.6 * arms[j] - 0.7 * gtab([row_c, adst[j]]) for j in range(NA)]

        p = jnp.zeros((L,), f32)
        for f in range(N):
            fcol = splat(f)
            w1f = gtab([splat(ROW_W1), fcol])
            w2f = gtab([splat(ROW_W2), fcol])
            wdf = gtab([splat(ROW_WD), fcol])
            cff = gtab([splat(ROW_CF), fcol])
            svf = gtab([splat(ROW_SV), fcol])
            t1 = jnp.zeros((L,), f32)
            t2 = jnp.zeros((L,), f32)
            for j in range(NA):
                o = (gtab([asrc[j], fcol]) + gtab([adst[j] + ROW_D, fcol])
                     + arms[j] * w1f + es[j] * w2f)
                t1 += o
                t2 += o * o
            for j in range(ND):
                o = gtab([dtgt[j] + ROW_T, fcol]) + darm[j] * wdf
                t1 += o
                t2 += o * o
            mu = t1 * (1.0 / NORD)
            var = t2 * (1.0 / NORD) - mu * mu * svf + EPS
            p = p + cff * t1 * _sc_rsqrt(var)

        out_v[...] = p
        pltpu.sync_copy(out_v, stage_hbm.at[wid])
        plsc.subcore_barrier()
        pltpu.sync_copy(stage_hbm, all_v)
        prows = [all_v[k] for k in range(NW)]
        mx = prows[0]
        for k in range(1, NW):
            mx = jnp.maximum(mx, prows[k])
        m = jnp.max(mx)
        se = jnp.zeros((L,), f32)
        for k in range(NW):
            se += jnp.exp(prows[k] - m)
        s = jnp.sum(se)
        sv = jnp.full((L,), 1.0, f32) * s
        e = lax.shift_right_logical(plsc.bitcast(sv, i32), 23) - 127
        y = (e.astype(f32) + 0.5) * jnp.float32(0.6931472)
        for _ in range(4):
            y = y + sv * jnp.exp(-y) - 1.0
        out_v[...] = p - m - y
        pltpu.sync_copy(out_v, p_hbm.at[wid])


def kernel(x1, x2, edges, at_src, at_dst, at_armies, dep_tgt, dep_armies, params):
    src_row = jnp.concatenate([edges[0], edges[1]]).astype(i32).reshape(1, E)
    dst_row = jnp.concatenate([edges[1], edges[0]]).astype(i32).reshape(1, E)

    p = params
    g1, g2 = p["g1"], p["g2"]
    r = lambda a: a.reshape(1, -1)
    tbl, v = pl.pallas_call(
        _tc1_body,
        out_shape=[
            jax.ShapeDtypeStruct((TBL_ROWS, N), f32),
            jax.ShapeDtypeStruct((1, 1), f32),
        ],
    )(x1.astype(f32), r(x2.astype(f32)), src_row, dst_row,
      src_row.reshape(E, 1), dst_row.reshape(E, 1),
      g1["Wq"], r(g1["bq"]), g1["Wk"], r(g1["bk"]), g1["Wv"], r(g1["bv"]),
      g1["Ws"], r(g1["bs"]), g1["Wb"],
      g2["Wq"], r(g2["bq"]), g2["Wk"], r(g2["bk"]), g2["Wv"], r(g2["bv"]),
      g2["Ws"], r(g2["bs"]), g2["Wb"],
      r(p["n1"]["w"]), r(p["n1"]["b"]), r(p["n1"]["s"]),
      r(p["n2"]["w"]), r(p["n2"]["b"]), r(p["n2"]["s"]),
      r(p["n3"]["w"]), r(p["n3"]["b"]), r(p["n3"]["s"]),
      p["gate_W"], r(p["gate_b"]), p["nn_W"], r(p["nn_b"]),
      p["lin1_W"], r(p["lin1_b"]),
      p["atk_W"], r(p["atk_b"]), p["dep_W"], r(p["dep_b"]),
      r(p["acc_W"]), r(p["n4"]["w"]), r(p["n4"]["s"]))

    logp = _sc_moves_kernel()(tbl, at_src.astype(i32), at_dst.astype(i32),
                              at_armies.astype(f32), dep_tgt.astype(i32),
                              dep_armies.astype(f32))
    return v.reshape(()), logp.reshape(M)


# speedup vs baseline: 4.4352x; 1.0121x over previous
import functools

import jax
import jax.numpy as jnp
from jax import lax
from jax.experimental import pallas as pl
from jax.experimental.pallas import tpu as pltpu
from jax.experimental.pallas import tpu_sc as plsc

f32 = jnp.float32
i32 = jnp.int32

N = 20
E = 200
M = 256
NA = 8
ND = 4
NW = 16
L = 16
NORD = float(NA + ND)
EPS = 1e-5

ROW_D = 20
ROW_T = 40
ROW_W1 = 60
ROW_W2 = 61
ROW_WD = 62
ROW_CF = 63
ROW_SV = 64
ROW_C = 65
TBL_ROWS = 72

MV_COLS = 32

_PIECES = [
    ("g1", 4 * 15 + 4 + 3), ("g2", 4 * 35 + 4 + 3),
    ("n1w", 1), ("n1b", 1), ("n1s", 1),
    ("n2w", 1), ("n2b", 1), ("n2s", 1),
    ("n3w", 1), ("n3b", 1), ("n3s", 1),
    ("gate", 3), ("nnW", 55), ("nnb", 1),
    ("l1a", 1), ("l1b", 1), ("scal", 1),
    ("atkW", 132), ("atkb", 1), ("depW", 66), ("depb", 1),
    ("acc", 1), ("n4w", 1), ("n4s", 1),
]
_OFF = {}
_r = 0
for _n, _sz in _PIECES:
    _OFF[_n] = _r
    _r += _sz
P_ROWS = _r


def _graph_norm_nodes(x, w, b, s):
    mean = jnp.mean(x, axis=0, keepdims=True)
    out = x - mean * s
    var = jnp.mean(out * out, axis=0, keepdims=True)
    return w * out / jnp.sqrt(var + EPS) + b


def _tconv(x, P, o, esrc_t, edst_t, valid_row):
    din = x.shape[1]
    Wq = P[o:o + din]
    bq = P[o + din:o + din + 1]
    o2 = o + din + 1
    Wk = P[o2:o2 + din]
    bk = P[o2 + din:o2 + din + 1]
    o3 = o2 + din + 1
    Wv = P[o3:o3 + din]
    bv = P[o3 + din:o3 + din + 1]
    o4 = o3 + din + 1
    Ws = P[o4:o4 + din]
    bs = P[o4 + din:o4 + din + 1]
    ow = o4 + din + 1
    wb = P[ow:ow + 3]
    q = jnp.dot(x, Wq, preferred_element_type=f32) + bq
    k = jnp.dot(x, Wk, preferred_element_type=f32) + bk
    v = jnp.dot(x, Wv, preferred_element_type=f32) + bv
    xr = jnp.dot(x, Ws, preferred_element_type=f32) + bs
    P2 = lax.dot_general(q, k, (((1,), (1,)), ((), ())), preferred_element_type=f32)
    s_row = jnp.sum(edst_t * jnp.dot(P2, esrc_t, preferred_element_type=f32),
                    axis=0, keepdims=True) * (1.0 / jnp.sqrt(f32(N)))
    edst_b = edst_t > 0.5
    masked = jnp.where(edst_b & valid_row, s_row, f32(-1e30))
    smax = jnp.max(masked, axis=1, keepdims=True)
    smax = jnp.where(smax > f32(-1e29), smax, f32(0.0))
    sm_e = jnp.sum(edst_t * smax, axis=0, keepdims=True)
    ex = jnp.where(valid_row, jnp.exp(s_row - sm_e), f32(0.0))
    den = jnp.sum(edst_t * ex, axis=1, keepdims=True)
    den_e = jnp.sum(edst_t * den, axis=0, keepdims=True)
    alpha = ex / den_e
    vs_t = lax.dot_general(v, esrc_t, (((0,), (0,)), ((), ())),
                           preferred_element_type=f32)
    agg = lax.dot_general(edst_t, alpha * vs_t, (((1,), (1,)), ((), ())),
                          preferred_element_type=f32)
    logit = (jnp.sum(agg * wb[0:1], axis=1, keepdims=True)
             + jnp.sum(xr * wb[1:2], axis=1, keepdims=True)
             + jnp.sum((agg - xr) * wb[2:3], axis=1, keepdims=True))
    beta = jax.nn.sigmoid(logit)
    return beta * xr + (1.0 - beta) * agg


def _tc1_body(x1_ref, x2_ref, src_row_ref, dst_row_ref, src_col_ref,
              dst_col_ref, P_ref, tbl_ref, v_ref):
    x1 = x1_ref[...]
    P = P_ref[...]
    src_row = src_row_ref[...]
    dst_row = dst_row_ref[...]
    h_col = src_col_ref[...] * N + dst_col_ref[...]
    h_row = src_row * N + dst_row
    eq = h_col == h_row
    ii = lax.broadcasted_iota(i32, (E, E), 0)
    jj = lax.broadcasted_iota(i32, (E, E), 1)
    dup_row = jnp.any(eq & (ii < jj), axis=0, keepdims=True)
    valid_row = ~dup_row
    nodes_col = lax.broadcasted_iota(i32, (N, 1), 0)
    esrc_t = jnp.where(nodes_col == src_row, f32(1.0), f32(0.0))
    edst_t = jnp.where(nodes_col == dst_row, f32(1.0), f32(0.0))

    O = _OFF
    xa = _graph_norm_nodes(
        jnp.maximum(_tconv(x1, P, O["g1"], esrc_t, edst_t, valid_row), 0.0),
        P[O["n1w"]:O["n1w"] + 1], P[O["n1b"]:O["n1b"] + 1], P[O["n1s"]:O["n1s"] + 1])
    xb = _graph_norm_nodes(
        jnp.maximum(_tconv(jnp.concatenate([x1, xa], axis=1), P, O["g2"],
                           esrc_t, edst_t, valid_row), 0.0),
        P[O["n2w"]:O["n2w"] + 1], P[O["n2b"]:O["n2b"] + 1], P[O["n2s"]:O["n2s"] + 1])
    xc = jnp.concatenate([x1, xa, xb], axis=1)

    scal = P[O["scal"]:O["scal"] + 1]
    gate_b = scal[:, 0:1]
    lin1_b = scal[:, 1:2]
    gw = P[O["gate"]:O["gate"] + 3]
    gl = (jnp.sum(x1 * gw[0:1, 0:15], axis=1, keepdims=True)
          + jnp.sum(xa * gw[1:2], axis=1, keepdims=True)
          + jnp.sum(xb * gw[2:3], axis=1, keepdims=True) + gate_b)
    gl = gl - jnp.max(gl)
    g = jnp.exp(gl)
    g = g / jnp.sum(g)
    h = jnp.dot(xc, P[O["nnW"]:O["nnW"] + 55], preferred_element_type=f32) \
        + P[O["nnb"]:O["nnb"] + 1]
    xg = jnp.sum(g * h, axis=0, keepdims=True)
    n3s = P[O["n3s"]:O["n3s"] + 1]
    mg = jnp.mean(xg, axis=1, keepdims=True)
    outg = xg - mg * n3s
    varg = jnp.mean(outg * outg, axis=1, keepdims=True)
    xgn = P[O["n3w"]:O["n3w"] + 1] * outg / jnp.sqrt(varg + EPS) \
        + P[O["n3b"]:O["n3b"] + 1]
    val = (jnp.sum(jnp.maximum(xgn, 0.0) * P[O["l1a"]:O["l1a"] + 1],
                   axis=1, keepdims=True)
           + jnp.sum(x2_ref[...] * P[O["l1b"]:O["l1b"] + 1][:, 0:4],
                     axis=1, keepdims=True) + lin1_b)
    v_ref[...] = jnp.tanh(val)

    mask = x1[:, 5:15] == f32(1.0)
    col_sum = jnp.sum(jnp.where(mask, x1[:, 0:1], 0.0), axis=0, keepdims=True)
    col_cnt = jnp.sum(jnp.where(mask, f32(1.0), f32(0.0)), axis=0, keepdims=True)
    den = jnp.where(col_cnt - 1.0 > 0.0, col_cnt - 1.0, 1.0)
    tmp = jnp.where(mask, (col_sum - x1[:, 0:1]) / den, 0.0)

    X = jnp.concatenate([xa, xb], axis=1)
    F = jnp.concatenate([x1, tmp, X], axis=1)
    atkW = P[O["atkW"]:O["atkW"] + 132]
    depW = P[O["depW"]:O["depW"] + 66]
    Wsrc = jnp.concatenate([atkW[0:15], atkW[30:40], atkW[50:90]], axis=0)
    Wdst = jnp.concatenate([atkW[15:30], atkW[40:50], atkW[90:130]], axis=0)
    Sp = jnp.dot(F, Wsrc, preferred_element_type=f32) + P[O["atkb"]:O["atkb"] + 1]
    Dd = jnp.dot(F, Wdst, preferred_element_type=f32)
    Tp = jnp.dot(F, depW[0:65], preferred_element_type=f32) + P[O["depb"]:O["depb"] + 1]
    pick = jnp.where((lax.broadcasted_iota(i32, (1, 15), 1) == 3)
                     | (lax.broadcasted_iota(i32, (1, 15), 1) == 4),
                     f32(1.0), f32(0.0))
    c_row = lax.dot_general(pick, x1, (((1,), (1,)), ((), ())),
                            preferred_element_type=f32)
    acc = P[O["acc"]:O["acc"] + 1]
    n4w = P[O["n4w"]:O["n4w"] + 1]
    n4s = P[O["n4s"]:O["n4s"] + 1]
    cf_row = n4w * acc * (1.0 - n4s)
    sv_row = n4s * (2.0 - n4s)
    tbl_ref[...] = jnp.concatenate(
        [Sp, Dd, Tp, atkW[130:131], atkW[131:132], depW[65:66], cf_row, sv_row,
         c_row, jnp.zeros((TBL_ROWS - ROW_C - 1, N), f32)], axis=0)


def _sc_rsqrt(x):
    i = plsc.bitcast(x, i32)
    i = jnp.int32(0x5F3759DF) - lax.shift_right_logical(i, 1)
    y = plsc.bitcast(i, f32)
    for _ in range(3):
        y = y * (1.5 - 0.5 * x * y * y)
    return y


@functools.cache
def _sc_moves_kernel():
    mesh = plsc.VectorSubcoreMesh(core_axis_name="c", subcore_axis_name="s",
                                  num_cores=2, num_subcores=16)
    return pl.kernel(
        _sc_moves_body,
        out_type=jax.ShapeDtypeStruct((NW, L), f32),
        mesh=mesh,
        compiler_params=pltpu.CompilerParams(needs_layout_passes=False),
        scratch_types=[
            pltpu.VMEM((TBL_ROWS, N), f32),
            pltpu.VMEM((L, MV_COLS), i32),
            pltpu.VMEM((L,), f32),
            pltpu.VMEM((NW, L), f32),
            pltpu.HBM((NW, L), f32),
            pltpu.SemaphoreType.DMA,
        ],
    )


def _sc_moves_body(tbl_hbm, mv_hbm, p_hbm, tbl_v, mv_v, out_v, all_v,
                   stage_hbm, sem):
    wid = lax.axis_index("s")

    @pl.when(lax.axis_index("c") == 0)
    def _():
        c1 = pltpu.async_copy(tbl_hbm, tbl_v, sem)
        c2 = pltpu.async_copy(mv_hbm.at[pl.ds(wid * L, L)], mv_v, sem)
        c1.wait()
        c2.wait()

        def splat(v):
            return jnp.full((L,), v, i32)

        lane = jnp.arange(L, dtype=i32)
        gtab = functools.partial(plsc.load_gather, tbl_v)

        def mvcol(j):
            return plsc.load_gather(mv_v, [lane, splat(j)])

        asrc = [mvcol(j) for j in range(NA)]
        adst = [mvcol(NA + j) for j in range(NA)]
        arms = [plsc.bitcast(mvcol(2 * NA + j), f32) for j in range(NA)]
        dtgt = [mvcol(3 * NA + j) for j in range(ND)]
        darm = [plsc.bitcast(mvcol(3 * NA + ND + j), f32) for j in range(ND)]
        row_c = splat(ROW_C)
        es = [0.6 * arms[j] - 0.7 * gtab([row_c, adst[j]]) for j in range(NA)]

        p = jnp.zeros((L,), f32)
        for f in range(N):
            fcol = splat(f)
            w1f = gtab([splat(ROW_W1), fcol])
            w2f = gtab([splat(ROW_W2), fcol])
            wdf = gtab([splat(ROW_WD), fcol])
            cff = gtab([splat(ROW_CF), fcol])
            svf = gtab([splat(ROW_SV), fcol])
            t1 = jnp.zeros((L,), f32)
            t2 = jnp.zeros((L,), f32)
            for j in range(NA):
                o = (gtab([asrc[j], fcol]) + gtab([adst[j] + ROW_D, fcol])
                     + arms[j] * w1f + es[j] * w2f)
                t1 += o
                t2 += o * o
            for j in range(ND):
                o = gtab([dtgt[j] + ROW_T, fcol]) + darm[j] * wdf
                t1 += o
                t2 += o * o
            mu = t1 * (1.0 / NORD)
            var = t2 * (1.0 / NORD) - mu * mu * svf + EPS
            p = p + cff * t1 * _sc_rsqrt(var)

        out_v[...] = p
        pltpu.sync_copy(out_v, stage_hbm.at[wid])
        plsc.subcore_barrier()
        pltpu.sync_copy(stage_hbm, all_v)
        prows = [all_v[k] for k in range(NW)]
        mx = prows[0]
        for k in range(1, NW):
            mx = jnp.maximum(mx, prows[k])
        m = jnp.max(mx)
        se = jnp.zeros((L,), f32)
        for k in range(NW):
            se += jnp.exp(prows[k] - m)
        s = jnp.sum(se)
        sv = jnp.full((L,), 1.0, f32) * s
        e = lax.shift_right_logical(plsc.bitcast(sv, i32), 23) - 127
        y = (e.astype(f32) + 0.5) * jnp.float32(0.6931472)
        for _ in range(4):
            y = y + sv * jnp.exp(-y) - 1.0
        out_v[...] = p - m - y
        pltpu.sync_copy(out_v, p_hbm.at[wid])


def kernel(x1, x2, edges, at_src, at_dst, at_armies, dep_tgt, dep_armies, params):
    src_cat = jnp.concatenate([edges[0], edges[1]]).astype(i32)
    dst_cat = jnp.concatenate([edges[1], edges[0]]).astype(i32)

    p = params
    g1, g2 = p["g1"], p["g2"]
    fl = lambda a: a.astype(f32).reshape(-1)
    z5 = jnp.zeros((5,), f32)
    gw = fl(p["gate_W"])
    pieces = {
        "g1": [fl(g1["Wq"]), fl(g1["bq"]), fl(g1["Wk"]), fl(g1["bk"]),
               fl(g1["Wv"]), fl(g1["bv"]), fl(g1["Ws"]), fl(g1["bs"]),
               fl(g1["Wb"])],
        "g2": [fl(g2["Wq"]), fl(g2["bq"]), fl(g2["Wk"]), fl(g2["bk"]),
               fl(g2["Wv"]), fl(g2["bv"]), fl(g2["Ws"]), fl(g2["bs"]),
               fl(g2["Wb"])],
        "n1w": [fl(p["n1"]["w"])], "n1b": [fl(p["n1"]["b"])], "n1s": [fl(p["n1"]["s"])],
        "n2w": [fl(p["n2"]["w"])], "n2b": [fl(p["n2"]["b"])], "n2s": [fl(p["n2"]["s"])],
        "n3w": [fl(p["n3"]["w"])], "n3b": [fl(p["n3"]["b"])], "n3s": [fl(p["n3"]["s"])],
        "gate": [gw[0:15], z5, gw[15:55]],
        "nnW": [fl(p["nn_W"])], "nnb": [fl(p["nn_b"])],
        "l1a": [fl(p["lin1_W"])[0:20]],
        "l1b": [fl(p["lin1_W"])[20:24], jnp.zeros((16,), f32)],
        "scal": [fl(p["gate_b"]), fl(p["lin1_b"]), jnp.zeros((18,), f32)],
        "atkW": [fl(p["atk_W"])], "atkb": [fl(p["atk_b"])],
        "depW": [fl(p["dep_W"])], "depb": [fl(p["dep_b"])],
        "acc": [fl(p["acc_W"])], "n4w": [fl(p["n4"]["w"])], "n4s": [fl(p["n4"]["s"])],
    }
    flat = jnp.concatenate([a for name, _ in _PIECES for a in pieces[name]])
    P = flat.reshape(P_ROWS, N)

    tbl, v = pl.pallas_call(
        _tc1_body,
        out_shape=[
            jax.ShapeDtypeStruct((TBL_ROWS, N), f32),
            jax.ShapeDtypeStruct((1, 1), f32),
        ],
    )(x1.astype(f32), x2.astype(f32).reshape(1, 4), src_cat.reshape(1, E),
      dst_cat.reshape(1, E), src_cat.reshape(E, 1), dst_cat.reshape(E, 1), P)

    bits = lambda a: lax.bitcast_convert_type(a.astype(f32), i32)
    mv = jnp.concatenate(
        [at_src.astype(i32), at_dst.astype(i32), bits(at_armies),
         dep_tgt.astype(i32), bits(dep_armies)], axis=1)

    logp = _sc_moves_kernel()(tbl, mv)
    return v.reshape(()), logp.reshape(M)
